# Initial kernel scaffold; baseline (speedup 1.0000x reference)
import jax
import jax.numpy as jnp
from jax.experimental import pallas as pl


def kernel(src, seg, edge_index, src_table, seg_table, w):
    raise NotImplementedError("write your pallas kernel here")


# speedup vs baseline: 2.9499x; 2.9499x over previous
import functools

import jax
import jax.numpy as jnp
import numpy as np
from jax import lax
from jax.experimental import pallas as pl
from jax.experimental.pallas import tpu as pltpu
from jax.experimental.pallas import tpu_sc as plsc

EMB = 64
SEQ = 200
NCOL = 4 * EMB
VOCAB_HALF = 500000

_ii, _dd, _ee = np.meshgrid(
    np.arange(EMB), np.arange(4), np.arange(EMB), indexing="ij"
)
_tt = (25 * _ii) // 8 + _dd
_jj = 64 * _dd - 8 * (_ii % 8) + _ee
_valid = (_jj >= 0) & (_jj < SEQ)
assert int(_valid.sum()) == EMB * SEQ
_IV = _ii[_valid]
_DV = _dd[_valid]
_EV = _ee[_valid]
_TV = _tt[_valid]
_JV = _jj[_valid]
_COLV = 4 * _IV + _DV

_EMAT = np.zeros((SEQ, NCOL), np.float32)
_EMAT[_tt[:, :, 0].ravel(), (4 * _ii + _dd)[:, :, 0].ravel()] = 1.0
_QMAT = np.zeros((EMB, SEQ * SEQ), np.float32)
_QMAT[_EV, _TV * SEQ + _JV] = 1.0
_UMAT = np.zeros((EMB, SEQ * NCOL), np.float32)
_UMAT[_EV, _JV * NCOL + _COLV] = 1.0
_VMAT = np.zeros((NCOL, EMB), np.float32)
_VMAT[np.arange(NCOL), np.arange(NCOL) // 4] = 1.0

_EPS = 1e-12

_NC = 2
_NS = 16
_NW = _NC * _NS
_CHUNK = 128


def _sc_gather_body(table_hbm, idx_hbm, out_hbm, idx_v, rows0, rows1,
                    gsem0, gsem1, ssem0, ssem1, *, n_chunks, per_w):
    wid = lax.axis_index("s") * _NC + lax.axis_index("c")
    base = wid * per_w
    pltpu.sync_copy(idx_hbm.at[wid], idx_v)

    rows = (rows0, rows1)
    gsems = (gsem0, gsem1)
    ssems = (ssem0, ssem1)

    pltpu.async_copy(table_hbm.at[idx_v.at[0]], rows0, gsem0)

    def step(k0, _):
        for b in range(2):
            k = 2 * k0 + b
            o = 1 - b
            pltpu.make_async_copy(
                table_hbm.at[idx_v.at[k]], rows[b], gsems[b]).wait()
            pltpu.async_copy(
                rows[b], out_hbm.at[pl.ds(base + k * _CHUNK, _CHUNK)],
                ssems[b])

            @pl.when(k >= 1)
            def _wait_prev_scatter():
                pltpu.make_async_copy(
                    rows[o], out_hbm.at[pl.ds(base, _CHUNK)], ssems[o]).wait()

            @pl.when(k + 1 < n_chunks)
            def _fire_next_gather():
                pltpu.async_copy(
                    table_hbm.at[idx_v.at[k + 1]], rows[o], gsems[o])
        return 0

    lax.fori_loop(0, n_chunks // 2, step, 0)
    pltpu.make_async_copy(
        rows[1], out_hbm.at[pl.ds(base, _CHUNK)], ssems[1]).wait()


def _sc_gather(table, idx_flat):
    n = idx_flat.shape[0]
    per_w = n // _NW
    n_chunks = per_w // _CHUNK
    assert per_w * _NW == n and n_chunks * _CHUNK == per_w
    assert n_chunks % 2 == 0
    idx3 = idx_flat.reshape(_NW, n_chunks, _CHUNK)
    mesh = plsc.VectorSubcoreMesh(core_axis_name="c", subcore_axis_name="s")
    body = functools.partial(_sc_gather_body, n_chunks=n_chunks, per_w=per_w)
    run = pl.kernel(
        body,
        out_type=jax.ShapeDtypeStruct((n, EMB), jnp.float32),
        mesh=mesh,
        compiler_params=pltpu.CompilerParams(use_tc_tiling_on_sc=False),
        scratch_types=[
            pltpu.VMEM((n_chunks, _CHUNK), jnp.int32),
            pltpu.VMEM((_CHUNK, EMB), jnp.float32),
            pltpu.VMEM((_CHUNK, EMB), jnp.float32),
            pltpu.SemaphoreType.DMA,
            pltpu.SemaphoreType.DMA,
            pltpu.SemaphoreType.DMA,
            pltpu.SemaphoreType.DMA,
        ],
    )
    return run(table, idx3)


def _tc_body(ms_ref, seg_ref, w_ref, q_ref, u_ref, e_ref, v_ref, o_ref):
    hi = jax.lax.Precision.HIGHEST
    ms = ms_ref[...]
    w = w_ref[...]
    msw = ms * w[0][None, None, :]
    n2s = jnp.sum(msw * msw, axis=1)
    inv_s = 1.0 / jnp.maximum(jnp.sqrt(n2s), _EPS)
    out_s = jnp.sum(msw * inv_s[:, None, :], axis=2)

    seg = seg_ref[...]
    ohs = []
    n2g = jnp.zeros(n2s.shape, jnp.float32)
    for k in range(3):
        oh = (seg == k).astype(jnp.float32)
        ohs.append(oh)
        n2g = n2g + jnp.dot(oh, q_ref[k], precision=hi,
                            preferred_element_type=jnp.float32)
    inv_g = 1.0 / jnp.maximum(jnp.sqrt(n2g), _EPS)
    z = jnp.zeros((seg.shape[0], NCOL), jnp.float32)
    for k in range(3):
        s_k = jnp.dot(inv_g, u_ref[k], precision=hi,
                      preferred_element_type=jnp.float32)
        oh3 = jnp.dot(ohs[k], e_ref[...], precision=hi,
                      preferred_element_type=jnp.float32)
        z = z + oh3 * s_k
    out_g = jnp.dot(z, v_ref[...], precision=hi,
                    preferred_element_type=jnp.float32)
    o_ref[...] = out_s + out_g


def _tc_combine(ms, seg, w2, q, u, e, v, block_b):
    batch = seg.shape[0]
    n_blocks = batch // block_b
    grid = (n_blocks,)
    return pl.pallas_call(
        _tc_body,
        grid=grid,
        in_specs=[
            pl.BlockSpec((block_b, EMB, SEQ), lambda i: (i, 0, 0)),
            pl.BlockSpec((block_b, SEQ), lambda i: (i, 0)),
            pl.BlockSpec((1, SEQ), lambda i: (0, 0)),
            pl.BlockSpec((3, SEQ, SEQ), lambda i: (0, 0, 0)),
            pl.BlockSpec((3, SEQ, NCOL), lambda i: (0, 0, 0)),
            pl.BlockSpec((SEQ, NCOL), lambda i: (0, 0)),
            pl.BlockSpec((NCOL, EMB), lambda i: (0, 0)),
        ],
        out_specs=pl.BlockSpec((block_b, EMB), lambda i: (i, 0)),
        out_shape=jax.ShapeDtypeStruct((batch, EMB), jnp.float32),
    )(ms, seg, w2, q, u, e, v)


def kernel(src, seg, edge_index, src_table, seg_table, w):
    batch, seq = src.shape
    assert seq == SEQ and src_table.shape[1] == EMB

    t2 = src_table.reshape(VOCAB_HALF, 2 * EMB)
    t2 = jax.lax.optimization_barrier(t2)
    tlin = t2.reshape(src_table.shape)

    hi = jax.lax.Precision.HIGHEST
    tsq = seg_table * seg_table
    q = jnp.dot(tsq, jnp.asarray(_QMAT), precision=hi).reshape(3, SEQ, SEQ)
    u = jnp.dot(seg_table, jnp.asarray(_UMAT), precision=hi).reshape(
        3, SEQ, NCOL)
    emat = jnp.asarray(_EMAT)
    vmat = jnp.asarray(_VMAT)
    w2 = w.reshape(1, SEQ)

    n_slices = 4
    sb = batch // n_slices
    outs = []
    for s in range(n_slices):
        gbuf = _sc_gather(tlin, src[s * sb:(s + 1) * sb].reshape(-1))
        ms = gbuf.reshape(sb, EMB, SEQ)
        outs.append(_tc_combine(ms, seg[s * sb:(s + 1) * sb], w2,
                                q, u, emat, vmat, block_b=256))
    return jnp.concatenate(outs, axis=0)


# speedup vs baseline: 2.9526x; 1.0009x over previous
import functools

import jax
import jax.numpy as jnp
import numpy as np
from jax import lax
from jax.experimental import pallas as pl
from jax.experimental.pallas import tpu as pltpu
from jax.experimental.pallas import tpu_sc as plsc

EMB = 64
SEQ = 200
NCOL = 4 * EMB
VOCAB_HALF = 500000

_ii, _dd, _ee = np.meshgrid(
    np.arange(EMB), np.arange(4), np.arange(EMB), indexing="ij"
)
_tt = (25 * _ii) // 8 + _dd
_jj = 64 * _dd - 8 * (_ii % 8) + _ee
_valid = (_jj >= 0) & (_jj < SEQ)
assert int(_valid.sum()) == EMB * SEQ
_IV = _ii[_valid]
_DV = _dd[_valid]
_EV = _ee[_valid]
_TV = _tt[_valid]
_JV = _jj[_valid]
_COLV = 4 * _IV + _DV

_EMAT = np.zeros((SEQ, NCOL), np.float32)
_EMAT[_tt[:, :, 0].ravel(), (4 * _ii + _dd)[:, :, 0].ravel()] = 1.0
_QMAT = np.zeros((EMB, SEQ * SEQ), np.float32)
_QMAT[_EV, _TV * SEQ + _JV] = 1.0
_UMAT = np.zeros((EMB, SEQ * NCOL), np.float32)
_UMAT[_EV, _JV * NCOL + _COLV] = 1.0
_VMAT = np.zeros((NCOL, EMB), np.float32)
_VMAT[np.arange(NCOL), np.arange(NCOL) // 4] = 1.0

_EPS = 1e-12

_NC = 2
_NS = 16
_NW = _NC * _NS
_CHUNK = 128


def _sc_gather_body(table_hbm, idx_hbm, out_hbm, idx_v, rows0, rows1,
                    gsem0, gsem1, ssem0, ssem1, *, n_chunks, per_w):
    wid = lax.axis_index("s") * _NC + lax.axis_index("c")
    base = wid * per_w
    pltpu.sync_copy(idx_hbm.at[wid], idx_v)

    rows = (rows0, rows1)
    gsems = (gsem0, gsem1)
    ssems = (ssem0, ssem1)

    pltpu.async_copy(table_hbm.at[idx_v.at[0]], rows0, gsem0)

    def step(k0, _):
        for b in range(2):
            k = 2 * k0 + b
            o = 1 - b
            pltpu.make_async_copy(
                table_hbm.at[idx_v.at[k]], rows[b], gsems[b]).wait()
            pltpu.async_copy(
                rows[b], out_hbm.at[pl.ds(base + k * _CHUNK, _CHUNK)],
                ssems[b])

            @pl.when(k >= 1)
            def _wait_prev_scatter():
                pltpu.make_async_copy(
                    rows[o], out_hbm.at[pl.ds(base, _CHUNK)], ssems[o]).wait()

            @pl.when(k + 1 < n_chunks)
            def _fire_next_gather():
                pltpu.async_copy(
                    table_hbm.at[idx_v.at[k + 1]], rows[o], gsems[o])
        return 0

    lax.fori_loop(0, n_chunks // 2, step, 0)
    pltpu.make_async_copy(
        rows[1], out_hbm.at[pl.ds(base, _CHUNK)], ssems[1]).wait()


def _sc_gather(table, idx_flat):
    n = idx_flat.shape[0]
    per_w = n // _NW
    n_chunks = per_w // _CHUNK
    assert per_w * _NW == n and n_chunks * _CHUNK == per_w
    assert n_chunks % 2 == 0
    idx3 = idx_flat.reshape(_NW, n_chunks, _CHUNK)
    mesh = plsc.VectorSubcoreMesh(core_axis_name="c", subcore_axis_name="s")
    body = functools.partial(_sc_gather_body, n_chunks=n_chunks, per_w=per_w)
    run = pl.kernel(
        body,
        out_type=jax.ShapeDtypeStruct((n, EMB), jnp.float32),
        mesh=mesh,
        compiler_params=pltpu.CompilerParams(use_tc_tiling_on_sc=False),
        scratch_types=[
            pltpu.VMEM((n_chunks, _CHUNK), jnp.int32),
            pltpu.VMEM((_CHUNK, EMB), jnp.float32),
            pltpu.VMEM((_CHUNK, EMB), jnp.float32),
            pltpu.SemaphoreType.DMA,
            pltpu.SemaphoreType.DMA,
            pltpu.SemaphoreType.DMA,
            pltpu.SemaphoreType.DMA,
        ],
    )
    return run(table, idx3)


def _tc_body(ms_ref, seg_ref, w_ref, q_ref, u_ref, e_ref, v_ref, o_ref):
    hi = jax.lax.Precision.HIGHEST
    ms = ms_ref[...]
    w = w_ref[...]
    msw = ms * w[0][None, None, :]
    n2s = jnp.sum(msw * msw, axis=1)
    inv_s = 1.0 / jnp.maximum(jnp.sqrt(n2s), _EPS)
    out_s = jnp.sum(msw * inv_s[:, None, :], axis=2)

    seg = seg_ref[...]
    ohs = []
    n2g = jnp.zeros(n2s.shape, jnp.float32)
    for k in range(3):
        oh = (seg == k).astype(jnp.float32)
        ohs.append(oh)
        n2g = n2g + jnp.dot(oh, q_ref[k], precision=hi,
                            preferred_element_type=jnp.float32)
    inv_g = 1.0 / jnp.maximum(jnp.sqrt(n2g), _EPS)
    z = jnp.zeros((seg.shape[0], NCOL), jnp.float32)
    for k in range(3):
        s_k = jnp.dot(inv_g, u_ref[k], precision=hi,
                      preferred_element_type=jnp.float32)
        oh3 = jnp.dot(ohs[k], e_ref[...], precision=hi,
                      preferred_element_type=jnp.float32)
        z = z + oh3 * s_k
    out_g = jnp.dot(z, v_ref[...], precision=hi,
                    preferred_element_type=jnp.float32)
    o_ref[...] = out_s + out_g


def _tc_combine(ms, seg, w2, q, u, e, v, block_b):
    batch = seg.shape[0]
    n_blocks = batch // block_b
    grid = (n_blocks,)
    return pl.pallas_call(
        _tc_body,
        grid=grid,
        in_specs=[
            pl.BlockSpec((block_b, EMB, SEQ), lambda i: (i, 0, 0)),
            pl.BlockSpec((block_b, SEQ), lambda i: (i, 0)),
            pl.BlockSpec((1, SEQ), lambda i: (0, 0)),
            pl.BlockSpec((3, SEQ, SEQ), lambda i: (0, 0, 0)),
            pl.BlockSpec((3, SEQ, NCOL), lambda i: (0, 0, 0)),
            pl.BlockSpec((SEQ, NCOL), lambda i: (0, 0)),
            pl.BlockSpec((NCOL, EMB), lambda i: (0, 0)),
        ],
        out_specs=pl.BlockSpec((block_b, EMB), lambda i: (i, 0)),
        out_shape=jax.ShapeDtypeStruct((batch, EMB), jnp.float32),
    )(ms, seg, w2, q, u, e, v)


def kernel(src, seg, edge_index, src_table, seg_table, w):
    batch, seq = src.shape
    assert seq == SEQ and src_table.shape[1] == EMB

    t2 = src_table.reshape(VOCAB_HALF, 2 * EMB)
    t2 = jax.lax.optimization_barrier(t2)
    tlin = t2.reshape(src_table.shape)

    hi = jax.lax.Precision.HIGHEST
    tsq = seg_table * seg_table
    q = jnp.dot(tsq, jnp.asarray(_QMAT), precision=hi).reshape(3, SEQ, SEQ)
    u = jnp.dot(seg_table, jnp.asarray(_UMAT), precision=hi).reshape(
        3, SEQ, NCOL)
    emat = jnp.asarray(_EMAT)
    vmat = jnp.asarray(_VMAT)
    w2 = w.reshape(1, SEQ)

    n_slices = 4
    sb = batch // n_slices
    outs = []
    for s in range(n_slices):
        gbuf = _sc_gather(tlin, src[s * sb:(s + 1) * sb].reshape(-1))
        ms = gbuf.reshape(sb, EMB, SEQ)
        outs.append(_tc_combine(ms, seg[s * sb:(s + 1) * sb], w2,
                                q, u, emat, vmat, block_b=128))
    return jnp.concatenate(outs, axis=0)
